# baseline (speedup 1.0000x reference)
import jax
import jax.numpy as jnp
from jax import lax
from jax.experimental import pallas as pl
from jax.experimental.pallas import tpu as pltpu
from jax.experimental.pallas import tpu_sc as plsc

N = 10000
E = 320000
B = 128
NB = 80
BLK = NB * B
NBLK = 32
E_PAD = NBLK * BLK
DUMP = N
ACC_ROWS = 10112
DEG_PAD = 10240
IDXC = 40
NCH = NB // IDXC
R = 10000
GRID = N // R

_MESH = plsc.VectorSubcoreMesh(core_axis_name="c", subcore_axis_name="s")


def _fill(ref, rows, val, cols=128):
    v = jnp.full((16,), val, jnp.float32)

    def body(i, _):
        for k in range(cols // 16):
            ref[i, pl.ds(k * 16, 16)] = v
        return 0

    lax.fori_loop(0, rows, body, 0)


def _deg(dst_r):
    def body(dst_ref, out, dst_v, ones_v, zeros_v, deg_sp):
        c = lax.axis_index("c")
        s = lax.axis_index("s")
        wid = c * 16 + s
        _fill(ones_v, 1, 1.0, cols=B)

        def zbody(i, _):
            for k in range(8):
                zeros_v[pl.ds(i * 128 + k * 16, 16)] = jnp.zeros((16,), jnp.float32)
            return 0

        lax.fori_loop(0, 5, zbody, 0)
        pltpu.sync_copy(zeros_v, deg_sp.at[pl.ds(s * 640, 640)])
        plsc.subcore_barrier()
        pltpu.sync_copy(dst_ref.at[wid], dst_v)

        def body_b(b, _):
            pltpu.sync_copy(ones_v.at[0], deg_sp.at[dst_v.at[b]], add=True)
            return 0

        lax.fori_loop(0, NB, body_b, 0)
        plsc.subcore_barrier()
        pltpu.sync_copy(deg_sp.at[pl.ds(s * 640, 640)],
                        out.at[c].at[pl.ds(s * 640, 640)])

    return pl.kernel(
        body,
        out_type=jax.ShapeDtypeStruct((2, DEG_PAD), jnp.float32),
        mesh=_MESH,
        scratch_types=[
            pltpu.VMEM((NB, B), jnp.int32),
            pltpu.VMEM((1, B), jnp.float32),
            pltpu.VMEM((640,), jnp.float32),
            pltpu.VMEM_SHARED((DEG_PAD,), jnp.float32),
        ],
    )(dst_r)


def _make_seg(two_tables):
    NRING = 2

    def body(h_ref, src_ref, dst_ref, out, src_v, dst_v, rows_v, acc,
             *sems):
        c = lax.axis_index("c")
        s = lax.axis_index("s")
        _fill(rows_v.at[0], B, 0.0)
        for k in range(4):
            pltpu.sync_copy(rows_v.at[0], acc.at[pl.ds(s * 632 + 128 * k, 128)])
        pltpu.sync_copy(rows_v.at[0].at[pl.ds(0, 120)],
                        acc.at[pl.ds(s * 632 + 512, 120)])
        plsc.subcore_barrier()

        tbl = h_ref.at[c] if two_tables else h_ref
        nblk = 2 if two_tables else 1

        def gather(b, i):
            pltpu.async_copy(tbl.at[src_v.at[b]], rows_v.at[i], sems[i])

        def gather_wait(b, i):
            pltpu.make_async_copy(tbl.at[src_v.at[b]], rows_v.at[i],
                                  sems[i]).wait()

        for j in range(nblk):
            blk = 2 * s + j if two_tables else c * 16 + s
            for h in range(NCH):
                pltpu.sync_copy(src_ref.at[blk, h], src_v)
                pltpu.sync_copy(dst_ref.at[blk, h], dst_v)

                for i in range(NRING):
                    gather(i, i)

                def body_k(k, _):
                    for i in range(NRING):
                        b = NRING * k + i
                        gather_wait(b, i)
                        pltpu.sync_copy(rows_v.at[i], acc.at[dst_v.at[b]],
                                        add=True)
                        nxt = b + NRING

                        @pl.when(nxt < IDXC)
                        def _():
                            gather(nxt, i)
                    return 0

                lax.fori_loop(0, IDXC // NRING, body_k, 0)
        plsc.subcore_barrier()
        @pl.when(s < 15)
        def _():
            pltpu.sync_copy(acc.at[pl.ds(s * 632, 632)],
                            out.at[c].at[pl.ds(s * 632, 632)])

        @pl.when(s == 15)
        def _():
            pltpu.sync_copy(acc.at[pl.ds(15 * 632, 520)],
                            out.at[c].at[pl.ds(15 * 632, 520)])

    return pl.kernel(
        body,
        out_type=jax.ShapeDtypeStruct((2, N, 128), jnp.float32),
        mesh=_MESH,
        scratch_types=[
            pltpu.VMEM((IDXC, B), jnp.int32),
            pltpu.VMEM((IDXC, B), jnp.int32),
            pltpu.VMEM((NRING, B, 128), jnp.float32),
            pltpu.VMEM_SHARED((ACC_ROWS, 128), jnp.float32),
        ] + [pltpu.SemaphoreType.DMA] * NRING,
    )


def _tc_b(x, W1, ca, cb):
    def body(x_ref, w_ref, ca_ref, cb_ref, out_ref):
        dis = lax.rsqrt(1.0 + ca_ref[...] + cb_ref[...])
        hp = jnp.dot(x_ref[...], w_ref[...],
                     preferred_element_type=jnp.float32) * dis
        out_ref[0] = hp[:, :128]
        out_ref[1] = hp[:, 128:]

    return pl.pallas_call(
        body,
        grid=(GRID,),
        in_specs=[
            pl.BlockSpec((R, 128), lambda i: (i, 0)),
            pl.BlockSpec((128, 256), lambda i: (0, 0)),
            pl.BlockSpec((R, 1), lambda i: (i, 0)),
            pl.BlockSpec((R, 1), lambda i: (i, 0)),
        ],
        out_specs=pl.BlockSpec((2, R, 128), lambda i: (0, i, 0)),
        out_shape=jax.ShapeDtypeStruct((2, N, 128), jnp.float32),
    )(x, W1, ca, cb)


def _tc_d(seg, h, ca, cb, b1, W2):
    def body(seg_ref, h_ref, ca_ref, cb_ref, b1_ref, w_ref, out_ref):
        dis = lax.rsqrt(1.0 + ca_ref[...] + cb_ref[...])
        b1v = b1_ref[...]
        o0 = jnp.maximum((seg_ref[0] + h_ref[0]) * dis + b1v[:, :128], 0.0)
        o1 = jnp.maximum((seg_ref[1] + h_ref[1]) * dis + b1v[:, 128:], 0.0)
        w = w_ref[...]
        g = (jnp.dot(o0, w[:128], preferred_element_type=jnp.float32)
             + jnp.dot(o1, w[128:], preferred_element_type=jnp.float32)) * dis
        out_ref[...] = g

    return pl.pallas_call(
        body,
        grid=(GRID,),
        in_specs=[
            pl.BlockSpec((2, R, 128), lambda i: (0, i, 0)),
            pl.BlockSpec((2, R, 128), lambda i: (0, i, 0)),
            pl.BlockSpec((R, 1), lambda i: (i, 0)),
            pl.BlockSpec((R, 1), lambda i: (i, 0)),
            pl.BlockSpec((1, 256), lambda i: (0, 0)),
            pl.BlockSpec((256, 128), lambda i: (0, 0)),
        ],
        out_specs=pl.BlockSpec((R, 128), lambda i: (i, 0)),
        out_shape=jax.ShapeDtypeStruct((N, 128), jnp.float32),
    )(seg, h, ca, cb, b1, W2)


def _tc_f(seg2, g, ca, cb, b2):
    def body(seg_ref, g_ref, ca_ref, cb_ref, b2_ref, out_ref):
        dis = lax.rsqrt(1.0 + ca_ref[...] + cb_ref[...])
        t = (seg_ref[0] + seg_ref[1] + g_ref[...]) * dis + b2_ref[...]
        out_ref[...] = jnp.maximum(t, 0.0)

    return pl.pallas_call(
        body,
        grid=(GRID,),
        in_specs=[
            pl.BlockSpec((2, R, 128), lambda i: (0, i, 0)),
            pl.BlockSpec((R, 128), lambda i: (i, 0)),
            pl.BlockSpec((R, 1), lambda i: (i, 0)),
            pl.BlockSpec((R, 1), lambda i: (i, 0)),
            pl.BlockSpec((1, 128), lambda i: (0, 0)),
        ],
        out_specs=pl.BlockSpec((R, 128), lambda i: (i, 0)),
        out_shape=jax.ShapeDtypeStruct((N, 128), jnp.float32),
    )(seg2, g, ca, cb, b2)


def kernel(x, edge_index, W1, b1, W2, b2):
    src = edge_index[0].astype(jnp.int32)
    dst = edge_index[1].astype(jnp.int32)
    npad = E_PAD - E
    src_p = jnp.concatenate([src, jnp.arange(npad, dtype=jnp.int32) % N])
    dst_p = jnp.concatenate([dst, jnp.full((npad,), DUMP, jnp.int32)])
    src_r = src_p.reshape(NBLK, NCH, IDXC, B)
    dst_r = dst_p.reshape(NBLK, NCH, IDXC, B)
    dst_r3 = dst_p.reshape(NBLK, NB, B)

    cnt2 = _deg(dst_r3)
    ca = cnt2[0, :N].reshape(N, 1)
    cb = cnt2[1, :N].reshape(N, 1)

    h = _tc_b(x, W1, ca, cb)
    seg1 = _make_seg(True)(h, src_r, dst_r)
    g = _tc_d(seg1, h, ca, cb, b1.reshape(1, 256), W2)
    seg2 = _make_seg(False)(g, src_r, dst_r)
    return _tc_f(seg2, g, ca, cb, b2.reshape(1, 128))


# speedup vs baseline: 1.0085x; 1.0085x over previous
import jax
import jax.numpy as jnp
from jax import lax
from jax.experimental import pallas as pl
from jax.experimental.pallas import tpu as pltpu
from jax.experimental.pallas import tpu_sc as plsc

N = 10000
E = 320000
B = 128
NB = 80
BLK = NB * B
NBLK = 32
E_PAD = NBLK * BLK
DUMP = N
ACC_ROWS = 10112
DEG_PAD = 10240
IDXC = 40
NCH = NB // IDXC
R = 5000
GRID = N // R

_MESH = plsc.VectorSubcoreMesh(core_axis_name="c", subcore_axis_name="s")


def _fill(ref, rows, val, cols=128):
    v = jnp.full((16,), val, jnp.float32)

    def body(i, _):
        for k in range(cols // 16):
            ref[i, pl.ds(k * 16, 16)] = v
        return 0

    lax.fori_loop(0, rows, body, 0)


def _deg(dst_r):
    def body(dst_ref, out, dst_v, ones_v, zeros_v, deg_sp):
        c = lax.axis_index("c")
        s = lax.axis_index("s")
        wid = c * 16 + s
        _fill(ones_v, 1, 1.0, cols=B)

        def zbody(i, _):
            for k in range(8):
                zeros_v[pl.ds(i * 128 + k * 16, 16)] = jnp.zeros((16,), jnp.float32)
            return 0

        lax.fori_loop(0, 5, zbody, 0)
        pltpu.sync_copy(zeros_v, deg_sp.at[pl.ds(s * 640, 640)])
        plsc.subcore_barrier()
        pltpu.sync_copy(dst_ref.at[wid], dst_v)

        def body_b(b, _):
            pltpu.sync_copy(ones_v.at[0], deg_sp.at[dst_v.at[b]], add=True)
            return 0

        lax.fori_loop(0, NB, body_b, 0)
        plsc.subcore_barrier()
        pltpu.sync_copy(deg_sp.at[pl.ds(s * 640, 640)],
                        out.at[c].at[pl.ds(s * 640, 640)])

    return pl.kernel(
        body,
        out_type=jax.ShapeDtypeStruct((2, DEG_PAD), jnp.float32),
        mesh=_MESH,
        scratch_types=[
            pltpu.VMEM((NB, B), jnp.int32),
            pltpu.VMEM((1, B), jnp.float32),
            pltpu.VMEM((640,), jnp.float32),
            pltpu.VMEM_SHARED((DEG_PAD,), jnp.float32),
        ],
    )(dst_r)


def _make_seg(two_tables):
    NRING = 2

    def body(h_ref, src_ref, dst_ref, out, src_v, dst_v, rows_v, acc,
             *sems):
        c = lax.axis_index("c")
        s = lax.axis_index("s")
        _fill(rows_v.at[0], B, 0.0)
        for k in range(4):
            pltpu.sync_copy(rows_v.at[0], acc.at[pl.ds(s * 632 + 128 * k, 128)])
        pltpu.sync_copy(rows_v.at[0].at[pl.ds(0, 120)],
                        acc.at[pl.ds(s * 632 + 512, 120)])
        plsc.subcore_barrier()

        tbl = h_ref.at[c] if two_tables else h_ref
        nblk = 2 if two_tables else 1

        def gather(b, i):
            pltpu.async_copy(tbl.at[src_v.at[b]], rows_v.at[i], sems[i])

        def gather_wait(b, i):
            pltpu.make_async_copy(tbl.at[src_v.at[b]], rows_v.at[i],
                                  sems[i]).wait()

        for j in range(nblk):
            blk = 2 * s + j if two_tables else c * 16 + s
            for h in range(NCH):
                pltpu.sync_copy(src_ref.at[blk, h], src_v)
                pltpu.sync_copy(dst_ref.at[blk, h], dst_v)

                for i in range(NRING):
                    gather(i, i)

                def body_k(k, _):
                    for i in range(NRING):
                        b = NRING * k + i
                        gather_wait(b, i)
                        pltpu.sync_copy(rows_v.at[i], acc.at[dst_v.at[b]],
                                        add=True)
                        nxt = b + NRING

                        @pl.when(nxt < IDXC)
                        def _():
                            gather(nxt, i)
                    return 0

                lax.fori_loop(0, IDXC // NRING, body_k, 0)
        plsc.subcore_barrier()
        @pl.when(s < 15)
        def _():
            pltpu.sync_copy(acc.at[pl.ds(s * 632, 632)],
                            out.at[c].at[pl.ds(s * 632, 632)])

        @pl.when(s == 15)
        def _():
            pltpu.sync_copy(acc.at[pl.ds(15 * 632, 520)],
                            out.at[c].at[pl.ds(15 * 632, 520)])

    return pl.kernel(
        body,
        out_type=jax.ShapeDtypeStruct((2, N, 128), jnp.float32),
        mesh=_MESH,
        scratch_types=[
            pltpu.VMEM((IDXC, B), jnp.int32),
            pltpu.VMEM((IDXC, B), jnp.int32),
            pltpu.VMEM((NRING, B, 128), jnp.float32),
            pltpu.VMEM_SHARED((ACC_ROWS, 128), jnp.float32),
        ] + [pltpu.SemaphoreType.DMA] * NRING,
    )


def _tc_b(x, W1, ca, cb):
    def body(x_ref, w_ref, ca_ref, cb_ref, out_ref):
        dis = lax.rsqrt(1.0 + ca_ref[...] + cb_ref[...])
        hp = jnp.dot(x_ref[...], w_ref[...],
                     preferred_element_type=jnp.float32) * dis
        out_ref[0] = hp[:, :128]
        out_ref[1] = hp[:, 128:]

    return pl.pallas_call(
        body,
        grid=(GRID,),
        in_specs=[
            pl.BlockSpec((R, 128), lambda i: (i, 0)),
            pl.BlockSpec((128, 256), lambda i: (0, 0)),
            pl.BlockSpec((R, 1), lambda i: (i, 0)),
            pl.BlockSpec((R, 1), lambda i: (i, 0)),
        ],
        out_specs=pl.BlockSpec((2, R, 128), lambda i: (0, i, 0)),
        out_shape=jax.ShapeDtypeStruct((2, N, 128), jnp.float32),
    )(x, W1, ca, cb)


def _tc_d(seg, h, ca, cb, b1, W2):
    def body(seg_ref, h_ref, ca_ref, cb_ref, b1_ref, w_ref, out_ref):
        dis = lax.rsqrt(1.0 + ca_ref[...] + cb_ref[...])
        b1v = b1_ref[...]
        o0 = jnp.maximum((seg_ref[0] + h_ref[0]) * dis + b1v[:, :128], 0.0)
        o1 = jnp.maximum((seg_ref[1] + h_ref[1]) * dis + b1v[:, 128:], 0.0)
        w = w_ref[...]
        g = (jnp.dot(o0, w[:128], preferred_element_type=jnp.float32)
             + jnp.dot(o1, w[128:], preferred_element_type=jnp.float32)) * dis
        out_ref[...] = g

    return pl.pallas_call(
        body,
        grid=(GRID,),
        in_specs=[
            pl.BlockSpec((2, R, 128), lambda i: (0, i, 0)),
            pl.BlockSpec((2, R, 128), lambda i: (0, i, 0)),
            pl.BlockSpec((R, 1), lambda i: (i, 0)),
            pl.BlockSpec((R, 1), lambda i: (i, 0)),
            pl.BlockSpec((1, 256), lambda i: (0, 0)),
            pl.BlockSpec((256, 128), lambda i: (0, 0)),
        ],
        out_specs=pl.BlockSpec((R, 128), lambda i: (i, 0)),
        out_shape=jax.ShapeDtypeStruct((N, 128), jnp.float32),
    )(seg, h, ca, cb, b1, W2)


def _tc_f(seg2, g, ca, cb, b2):
    def body(seg_ref, g_ref, ca_ref, cb_ref, b2_ref, out_ref):
        dis = lax.rsqrt(1.0 + ca_ref[...] + cb_ref[...])
        t = (seg_ref[0] + seg_ref[1] + g_ref[...]) * dis + b2_ref[...]
        out_ref[...] = jnp.maximum(t, 0.0)

    return pl.pallas_call(
        body,
        grid=(GRID,),
        in_specs=[
            pl.BlockSpec((2, R, 128), lambda i: (0, i, 0)),
            pl.BlockSpec((R, 128), lambda i: (i, 0)),
            pl.BlockSpec((R, 1), lambda i: (i, 0)),
            pl.BlockSpec((R, 1), lambda i: (i, 0)),
            pl.BlockSpec((1, 128), lambda i: (0, 0)),
        ],
        out_specs=pl.BlockSpec((R, 128), lambda i: (i, 0)),
        out_shape=jax.ShapeDtypeStruct((N, 128), jnp.float32),
    )(seg2, g, ca, cb, b2)


def kernel(x, edge_index, W1, b1, W2, b2):
    src = edge_index[0].astype(jnp.int32)
    dst = edge_index[1].astype(jnp.int32)
    npad = E_PAD - E
    src_p = jnp.concatenate([src, jnp.arange(npad, dtype=jnp.int32) % N])
    dst_p = jnp.concatenate([dst, jnp.full((npad,), DUMP, jnp.int32)])
    src_r = src_p.reshape(NBLK, NCH, IDXC, B)
    dst_r = dst_p.reshape(NBLK, NCH, IDXC, B)
    dst_r3 = dst_p.reshape(NBLK, NB, B)

    cnt2 = _deg(dst_r3)
    ca = cnt2[0, :N].reshape(N, 1)
    cb = cnt2[1, :N].reshape(N, 1)

    h = _tc_b(x, W1, ca, cb)
    seg1 = _make_seg(True)(h, src_r, dst_r)
    g = _tc_d(seg1, h, ca, cb, b1.reshape(1, 256), W2)
    seg2 = _make_seg(False)(g, src_r, dst_r)
    return _tc_f(seg2, g, ca, cb, b2.reshape(1, 128))
